# baseline (speedup 1.0000x reference)
from math import pi

import jax
import jax.numpy as jnp
from jax import lax
from jax.experimental import pallas as pl

_B = 16384
_W = 1024
_POSITIVE_NUM = 8
_NUM_NEG = 24
_LAMBDA_FOV = 1.0

_LO_BITS = 953267991
_HI_BITS = 1065353216
_N_ITERS = 27

_BR = 512


def _fused_kernel(xmin_ref, fl9_ref, p_ref, fsg_ref, fl_ref, loss_ref):
    f32 = jnp.float32

    i3 = fl9_ref[...]
    z = jnp.zeros((_BR, 1), f32)
    o = jnp.ones((_BR, 1), f32)
    fl_ref[...] = jnp.concatenate(
        [i3[:, 0:3], z, i3[:, 3:6], z, i3[:, 6:9], z, z, z, z, o], axis=1)

    xmin = xmin_ref[...]
    colid = lax.broadcasted_iota(jnp.int32, (_BR, _W), 1)
    rel = (colid - xmin) & (_W - 1)
    pos = rel < _POSITIVE_NUM
    fsg_ref[...] = pos.astype(f32)

    p = p_ref[...]
    pt = jnp.where(pos, f32(0.0), p)
    bits = lax.bitcast_convert_type(pt, jnp.int32)

    lo = jnp.full((_BR, 1), _LO_BITS, jnp.int32)
    hi = jnp.full((_BR, 1), _HI_BITS, jnp.int32)
    for _ in range(_N_ITERS):
        mid = lo + ((hi - lo + 1) >> 1)
        cnt = jnp.sum((bits >= mid).astype(jnp.int32), axis=1, keepdims=True)
        ok = cnt >= _NUM_NEG
        lo = jnp.where(ok, mid, lo)
        hi = jnp.where(ok, hi, mid - 1)

    kbits = lo
    kval = lax.bitcast_convert_type(kbits, f32)

    ln1mp = jnp.log1p(-p)
    sel = bits > kbits
    ngt = jnp.sum(jnp.where(sel, f32(1.0), f32(0.0)), axis=1, keepdims=True)
    sneg = jnp.sum(jnp.where(sel, ln1mp, f32(0.0)), axis=1, keepdims=True)
    neg_part = -(sneg + (_NUM_NEG - ngt) * jnp.log1p(-kval))

    q = jnp.where(pos, p, f32(1.0))
    w = _W
    while w > 1:
        w //= 2
        q = q[:, :w] * q[:, w:2 * w]
    pos_part = -jnp.log(q)

    loss_ref[...] = jnp.sum(pos_part + neg_part, keepdims=True)[None]


def kernel(gt_sensor2_T_sensor1, gt_e_l, pred_e_l, pred_f_score):
    R_inv = jnp.linalg.inv(gt_sensor2_T_sensor1[:, :3, :3])
    a = jnp.matmul(pred_e_l[:, :3, :3], R_inv)
    yaw = jnp.arctan2(a[:, 1, 0], a[:, 0, 0])
    f_idx = (-yaw + pi) / (2.0 * pi) * _W
    xmin = (f_idx.astype(jnp.int32) - _POSITIVE_NUM // 2).reshape(_B, 1)
    fl9 = jnp.linalg.inv(jnp.matmul(gt_e_l[:, :3, :3], R_inv)).reshape(_B, 9)

    grid = _B // _BR
    fsg, fl, partial = pl.pallas_call(
        _fused_kernel,
        grid=(grid,),
        in_specs=[
            pl.BlockSpec((_BR, 1), lambda g: (g, 0)),
            pl.BlockSpec((_BR, 9), lambda g: (g, 0)),
            pl.BlockSpec((_BR, _W), lambda g: (g, 0)),
        ],
        out_specs=[
            pl.BlockSpec((_BR, _W), lambda g: (g, 0)),
            pl.BlockSpec((_BR, 16), lambda g: (g, 0)),
            pl.BlockSpec((1, 1, 1), lambda g: (g, 0, 0)),
        ],
        out_shape=[
            jax.ShapeDtypeStruct((_B, _W), jnp.float32),
            jax.ShapeDtypeStruct((_B, 16), jnp.float32),
            jax.ShapeDtypeStruct((grid, 1, 1), jnp.float32),
        ],
    )(xmin, fl9, pred_f_score)

    denom = jnp.float32(_B * (_POSITIVE_NUM + _NUM_NEG))
    loss = jnp.sum(partial) / denom * _LAMBDA_FOV
    return (loss, fsg, fl.reshape(_B, 4, 4))


# speedup vs baseline: 1.8071x; 1.8071x over previous
from math import pi

import jax
import jax.numpy as jnp
from jax import lax
from jax.experimental import pallas as pl

_B = 16384
_W = 1024
_POSITIVE_NUM = 8
_NUM_NEG = 24
_LAMBDA_FOV = 1.0

_LO_BITS = 953267991
_HI_BITS = 1065353216
_N_ITERS = 27

_BR = 512
_K_EXACT = 1024


def _c3(ref, i, j):
    c = 3 * i + j
    return ref[:, c:c + 1]


def _fused_kernel(xmin_ref, m_ref, p_ref, fsg_ref, fl_ref, score_ref, loss_ref):
    f32 = jnp.float32

    m = [[_c3(m_ref, i, j) for j in range(3)] for i in range(3)]
    a00 = m[1][1] * m[2][2] - m[1][2] * m[2][1]
    a01 = m[0][2] * m[2][1] - m[0][1] * m[2][2]
    a02 = m[0][1] * m[1][2] - m[0][2] * m[1][1]
    a10 = m[1][2] * m[2][0] - m[1][0] * m[2][2]
    a11 = m[0][0] * m[2][2] - m[0][2] * m[2][0]
    a12 = m[0][2] * m[1][0] - m[0][0] * m[1][2]
    a20 = m[1][0] * m[2][1] - m[1][1] * m[2][0]
    a21 = m[0][1] * m[2][0] - m[0][0] * m[2][1]
    a22 = m[0][0] * m[1][1] - m[0][1] * m[1][0]
    det = m[0][0] * a00 + m[0][1] * a10 + m[0][2] * a20
    idet = 1.0 / det

    inv = [a00 * idet, a01 * idet, a02 * idet,
           a10 * idet, a11 * idet, a12 * idet,
           a20 * idet, a21 * idet, a22 * idet]
    sc = jnp.abs(inv[0])
    for e in inv[1:]:
        sc = jnp.maximum(sc, jnp.abs(e))
    score_ref[...] = sc

    z = jnp.zeros((_BR, 1), f32)
    o = jnp.ones((_BR, 1), f32)
    fl_ref[...] = jnp.concatenate(
        [inv[0], inv[1], inv[2], z,
         inv[3], inv[4], inv[5], z,
         inv[6], inv[7], inv[8], z,
         z, z, z, o], axis=1)

    xmin = xmin_ref[...]
    colid = lax.broadcasted_iota(jnp.int32, (_BR, _W), 1)
    rel = (colid - xmin) & (_W - 1)
    pos = rel < _POSITIVE_NUM
    fsg_ref[...] = pos.astype(f32)

    p = p_ref[...]
    pt = jnp.where(pos, f32(0.0), p)
    bits = lax.bitcast_convert_type(pt, jnp.int32)

    lo = jnp.full((_BR, 1), _LO_BITS, jnp.int32)
    hi = jnp.full((_BR, 1), _HI_BITS, jnp.int32)
    for _ in range(_N_ITERS):
        mid = lo + ((hi - lo + 1) >> 1)
        cnt = jnp.sum((bits >= mid).astype(jnp.int32), axis=1, keepdims=True)
        ok = cnt >= _NUM_NEG
        lo = jnp.where(ok, mid, lo)
        hi = jnp.where(ok, hi, mid - 1)

    kbits = lo
    kval = lax.bitcast_convert_type(kbits, f32)

    ln1mp = jnp.log1p(-p)
    sel = bits > kbits
    ngt = jnp.sum(jnp.where(sel, f32(1.0), f32(0.0)), axis=1, keepdims=True)
    sneg = jnp.sum(jnp.where(sel, ln1mp, f32(0.0)), axis=1, keepdims=True)
    neg_part = -(sneg + (_NUM_NEG - ngt) * jnp.log1p(-kval))

    q = jnp.where(pos, p, f32(1.0))
    w = _W
    while w > 1:
        w //= 2
        q = q[:, :w] * q[:, w:2 * w]
    pos_part = -jnp.log(q)

    loss_ref[...] = jnp.sum(pos_part + neg_part, keepdims=True)[None]


def kernel(gt_sensor2_T_sensor1, gt_e_l, pred_e_l, pred_f_score):
    R_inv = jnp.linalg.inv(gt_sensor2_T_sensor1[:, :3, :3])
    a = jnp.matmul(pred_e_l[:, :3, :3], R_inv)
    yaw = jnp.arctan2(a[:, 1, 0], a[:, 0, 0])
    f_idx = (-yaw + pi) / (2.0 * pi) * _W
    xmin = (f_idx.astype(jnp.int32) - _POSITIVE_NUM // 2).reshape(_B, 1)
    M = jnp.matmul(gt_e_l[:, :3, :3], R_inv)

    grid = _B // _BR
    fsg, fl, score, partial = pl.pallas_call(
        _fused_kernel,
        grid=(grid,),
        in_specs=[
            pl.BlockSpec((_BR, 1), lambda g: (g, 0)),
            pl.BlockSpec((_BR, 9), lambda g: (g, 0)),
            pl.BlockSpec((_BR, _W), lambda g: (g, 0)),
        ],
        out_specs=[
            pl.BlockSpec((_BR, _W), lambda g: (g, 0)),
            pl.BlockSpec((_BR, 16), lambda g: (g, 0)),
            pl.BlockSpec((_BR, 1), lambda g: (g, 0)),
            pl.BlockSpec((1, 1, 1), lambda g: (g, 0, 0)),
        ],
        out_shape=[
            jax.ShapeDtypeStruct((_B, _W), jnp.float32),
            jax.ShapeDtypeStruct((_B, 16), jnp.float32),
            jax.ShapeDtypeStruct((_B, 1), jnp.float32),
            jax.ShapeDtypeStruct((grid, 1, 1), jnp.float32),
        ],
    )(xmin, M.reshape(_B, 9), pred_f_score)

    _, idx = lax.top_k(score.reshape(_B), _K_EXACT)
    fl_sel = jnp.linalg.inv(M[idx]).reshape(_K_EXACT, 9)
    cols = jnp.array([0, 1, 2, 4, 5, 6, 8, 9, 10], jnp.int32)
    fl = fl.at[idx[:, None], cols[None, :]].set(fl_sel)

    denom = jnp.float32(_B * (_POSITIVE_NUM + _NUM_NEG))
    loss = jnp.sum(partial) / denom * _LAMBDA_FOV
    return (loss, fsg, fl.reshape(_B, 4, 4))
